# baseline (speedup 1.0000x reference)
import functools

import jax
import jax.numpy as jnp
from jax import lax
from jax.experimental import pallas as pl
from jax.experimental.pallas import tpu as pltpu
from jax.experimental.pallas import tpu_sc as plsc

_CLS_TOKEN_ID = 2
_LANES = 16


@functools.lru_cache(maxsize=None)
def _build_pooler(B, S, D):
    n_id_chunks = S // _LANES
    TAIL = min(256, S)
    n_tail_chunks = TAIL // _LANES
    H = D // 2
    n_h_chunks = H // _LANES
    mesh = plsc.VectorSubcoreMesh(
        core_axis_name="c", subcore_axis_name="s", num_cores=1
    )

    @functools.partial(
        pl.kernel,
        out_type=jax.ShapeDtypeStruct((B, D), jnp.float32),
        mesh=mesh,
        scratch_types=[
            pltpu.VMEM((S,), jnp.int32),
            pltpu.VMEM((H,), jnp.float32),
            pltpu.VMEM((H,), jnp.float32),
            pltpu.VMEM((TAIL,), jnp.int32),
            pltpu.VMEM((_LANES,), jnp.int32),
            pltpu.SemaphoreType.DMA,
            pltpu.SemaphoreType.DMA,
        ],
    )
    def pooler(
        emb_hbm, ids_hbm, rf_hbm, out_hbm,
        ids_v, row_v, acc_v, tail_v, rf_v, sem_a, sem_b,
    ):
        wid = lax.axis_index("s")

        @pl.when(wid < 2 * B)
        def _tile_body():
            b = wid // 2
            half = wid % 2
            dlo = half * H

            rf_dma = pltpu.async_copy(rf_hbm, rf_v.at[pl.ds(0, 1)], sem_a)
            tail_dma = pltpu.async_copy(
                ids_hbm.at[b, pl.ds(S - TAIL, TAIL)], tail_v, sem_b
            )

            tail_dma.wait()
            lanes = lax.iota(jnp.int32, _LANES)
            best = jnp.full((_LANES,), -1, jnp.int32)
            for c in range(n_tail_chunks):
                v = tail_v[pl.ds(c * _LANES, _LANES)]
                pos = lanes + (S - TAIL + c * _LANES)
                best = jnp.maximum(
                    best, jnp.where(v == _CLS_TOKEN_ID, pos, -1)
                )
            tail_idx = best[0]
            for lane in range(1, _LANES):
                tail_idx = jnp.maximum(tail_idx, best[lane])

            def full_scan():
                pltpu.sync_copy(ids_hbm.at[b], ids_v)

                def scan_body(i, acc):
                    v = ids_v[pl.ds(i * _LANES, _LANES)]
                    pos = lanes + i * _LANES
                    return jnp.maximum(
                        acc, jnp.where(v == _CLS_TOKEN_ID, pos, -1)
                    )

                full = lax.fori_loop(
                    0,
                    n_id_chunks - n_tail_chunks,
                    scan_body,
                    jnp.full((_LANES,), -1, jnp.int32),
                )
                e = full[0]
                for lane in range(1, _LANES):
                    e = jnp.maximum(e, full[lane])
                return e

            end_idx = lax.cond(tail_idx >= 0, lambda: tail_idx, full_scan)

            rf_dma.wait()
            rf = rf_v[...][0]

            @pl.when(rf == 0)
            def _zero_out():
                for c in range(n_h_chunks):
                    row_v[pl.ds(c * _LANES, _LANES)] = jnp.zeros(
                        (_LANES,), jnp.float32
                    )
                pltpu.sync_copy(row_v, out_hbm.at[b, pl.ds(dlo, H)])

            @pl.when(jnp.logical_and(rf != 0, end_idx >= 0))
            def _gather_row():
                pltpu.sync_copy(
                    emb_hbm.at[b, end_idx, pl.ds(dlo, H)],
                    out_hbm.at[b, pl.ds(dlo, H)],
                )

            @pl.when(jnp.logical_and(rf != 0, end_idx < 0))
            def _weighted_mean():
                for c in range(n_h_chunks):
                    acc_v[pl.ds(c * _LANES, _LANES)] = jnp.zeros(
                        (_LANES,), jnp.float32
                    )

                def row_body(j, carry):
                    pltpu.sync_copy(
                        emb_hbm.at[b, j, pl.ds(dlo, H)], row_v
                    )
                    w = (j + 1).astype(jnp.float32)
                    for c in range(n_h_chunks):
                        sl = pl.ds(c * _LANES, _LANES)
                        acc_v[sl] = acc_v[sl] + row_v[sl] * w
                    return carry

                lax.fori_loop(0, S, row_body, 0)
                inv = jnp.float32(1.0 / S)
                for c in range(n_h_chunks):
                    sl = pl.ds(c * _LANES, _LANES)
                    acc_v[sl] = acc_v[sl] * inv
                pltpu.sync_copy(acc_v, out_hbm.at[b, pl.ds(dlo, H)])

    return pooler


def kernel(embeddings, input_ids, return_final):
    B, S, D = embeddings.shape
    pooler = _build_pooler(B, S, D)
    rf1 = jnp.reshape(jnp.asarray(return_final, jnp.int32), (1,))
    return pooler(
        embeddings.astype(jnp.float32), input_ids.astype(jnp.int32), rf1
    )


# speedup vs baseline: 1.0061x; 1.0061x over previous
import functools

import jax
import jax.numpy as jnp
from jax import lax
from jax.experimental import pallas as pl
from jax.experimental.pallas import tpu as pltpu
from jax.experimental.pallas import tpu_sc as plsc

_CLS_TOKEN_ID = 2
_LANES = 16


@functools.lru_cache(maxsize=None)
def _build_pooler(B, S, D):
    n_id_chunks = S // _LANES
    TAIL = min(256, S)
    n_tail_chunks = TAIL // _LANES
    H = D // 2
    n_h_chunks = H // _LANES
    mesh = plsc.VectorSubcoreMesh(
        core_axis_name="c", subcore_axis_name="s", num_cores=1
    )

    @functools.partial(
        pl.kernel,
        out_type=jax.ShapeDtypeStruct((B, D), jnp.float32),
        mesh=mesh,
        scratch_types=[
            pltpu.VMEM((S,), jnp.int32),
            pltpu.VMEM((H,), jnp.float32),
            pltpu.VMEM((H,), jnp.float32),
            pltpu.VMEM((TAIL,), jnp.int32),
            pltpu.VMEM((_LANES,), jnp.int32),
            pltpu.SemaphoreType.DMA,
            pltpu.SemaphoreType.DMA,
        ],
    )
    def pooler(
        emb_hbm, ids_hbm, rf_hbm, out_hbm,
        ids_v, row_v, acc_v, tail_v, rf_v, sem_a, sem_b,
    ):
        wid = lax.axis_index("s")

        @pl.when(wid < 2 * B)
        def _tile_body():
            b = wid // 2
            half = wid % 2
            dlo = half * H

            rf_dma = pltpu.async_copy(rf_hbm, rf_v.at[pl.ds(0, 1)], sem_a)
            tail_dma = pltpu.async_copy(
                ids_hbm.at[b, pl.ds(S - TAIL, TAIL)], tail_v, sem_b
            )

            tail_dma.wait()
            lanes = lax.iota(jnp.int32, _LANES)
            best = jnp.full((_LANES,), -1, jnp.int32)
            for c in range(n_tail_chunks):
                v = tail_v[pl.ds(c * _LANES, _LANES)]
                pos = lanes + (S - TAIL + c * _LANES)
                best = jnp.maximum(
                    best, jnp.where(v == _CLS_TOKEN_ID, pos, -1)
                )
            tail_idx = best[0]
            for lane in range(1, _LANES):
                tail_idx = jnp.maximum(tail_idx, best[lane])

            def full_scan():
                pltpu.sync_copy(ids_hbm.at[b], ids_v)

                def scan_body(i, acc):
                    v = ids_v[pl.ds(i * _LANES, _LANES)]
                    pos = lanes + i * _LANES
                    return jnp.maximum(
                        acc, jnp.where(v == _CLS_TOKEN_ID, pos, -1)
                    )

                full = lax.fori_loop(
                    0,
                    n_id_chunks - n_tail_chunks,
                    scan_body,
                    jnp.full((_LANES,), -1, jnp.int32),
                )
                e = full[0]
                for lane in range(1, _LANES):
                    e = jnp.maximum(e, full[lane])
                return e

            end_idx = lax.cond(tail_idx >= 0, lambda: tail_idx, full_scan)

            rf_dma.wait()
            rf = rf_v[...][0]

            @pl.when(rf == 0)
            def _zero_out():
                def zero_body(c, carry):
                    row_v[pl.ds(c * _LANES, _LANES)] = jnp.zeros(
                        (_LANES,), jnp.float32
                    )
                    return carry

                lax.fori_loop(0, n_h_chunks, zero_body, 0)
                pltpu.sync_copy(row_v, out_hbm.at[b, pl.ds(dlo, H)])

            @pl.when(jnp.logical_and(rf != 0, end_idx >= 0))
            def _gather_row():
                pltpu.sync_copy(
                    emb_hbm.at[b, end_idx, pl.ds(dlo, H)],
                    out_hbm.at[b, pl.ds(dlo, H)],
                )

            @pl.when(jnp.logical_and(rf != 0, end_idx < 0))
            def _weighted_mean():
                def zacc_body(c, carry):
                    acc_v[pl.ds(c * _LANES, _LANES)] = jnp.zeros(
                        (_LANES,), jnp.float32
                    )
                    return carry

                lax.fori_loop(0, n_h_chunks, zacc_body, 0)

                def row_body(j, carry):
                    pltpu.sync_copy(
                        emb_hbm.at[b, j, pl.ds(dlo, H)], row_v
                    )
                    w = (j + 1).astype(jnp.float32)

                    def acc_body(c, inner):
                        sl = pl.ds(c * _LANES, _LANES)
                        acc_v[sl] = acc_v[sl] + row_v[sl] * w
                        return inner

                    lax.fori_loop(0, n_h_chunks, acc_body, 0)
                    return carry

                lax.fori_loop(0, S, row_body, 0)
                inv = jnp.float32(1.0 / S)

                def scale_body(c, carry):
                    sl = pl.ds(c * _LANES, _LANES)
                    acc_v[sl] = acc_v[sl] * inv
                    return carry

                lax.fori_loop(0, n_h_chunks, scale_body, 0)
                pltpu.sync_copy(acc_v, out_hbm.at[b, pl.ds(dlo, H)])

    return pooler


def kernel(embeddings, input_ids, return_final):
    B, S, D = embeddings.shape
    pooler = _build_pooler(B, S, D)
    rf1 = jnp.reshape(jnp.asarray(return_final, jnp.int32), (1,))
    return pooler(
        embeddings.astype(jnp.float32), input_ids.astype(jnp.int32), rf1
    )
